# baseline (speedup 1.0000x reference)
import functools

import jax
import jax.numpy as jnp
import numpy as np
from jax import lax
from jax.experimental import pallas as pl
from jax.experimental.pallas import tpu as pltpu
from jax.experimental.pallas import tpu_sc as plsc

N = 10000
E = 320000
EMB = 128
NG = 64
BN_EPS = 1e-5

NC = 2
NS = 16
NW = NC * NS
EP = 327680
WE = EP // NW
CHS = 128
NCHS = WE // CHS
CHE = 80
NCHE = WE // CHE
NP = 10240
RPT = NP // NS

NB = 5
R = N // NB
BE = 2560

_f32 = jnp.float32


def _tc_pre0(attr8, w0):
    def body(a_ref, w0_ref, o0_ref):
        o0_ref[...] = jnp.maximum(
            jnp.dot(a_ref[...], w0_ref[...], preferred_element_type=_f32),
            0.0)

    return pl.pallas_call(
        body,
        grid=(EP // BE,),
        in_specs=[pl.BlockSpec((BE, 8), lambda i: (i, 0)),
                  pl.BlockSpec((8, EMB), lambda i: (0, 0))],
        out_specs=pl.BlockSpec((BE, EMB), lambda i: (i, 0)),
        out_shape=jax.ShapeDtypeStruct((EP, EMB), _f32),
    )(attr8, w0)


def _tc_pre12(attr8, w1, w2):
    def body(a_ref, w1_ref, w2_ref, o1_ref, o2_ref):
        a = a_ref[...]
        o1_ref[...] = jnp.dot(a, w1_ref[...], preferred_element_type=_f32)
        o2_ref[...] = jnp.dot(a, w2_ref[...], preferred_element_type=_f32)

    wspec = pl.BlockSpec((8, EMB), lambda i: (0, 0))
    return pl.pallas_call(
        body,
        grid=(EP // BE,),
        in_specs=[pl.BlockSpec((BE, 8), lambda i: (i, 0)), wspec, wspec],
        out_specs=[pl.BlockSpec((BE, EMB), lambda i: (i, 0))] * 2,
        out_shape=[jax.ShapeDtypeStruct((EP, EMB), _f32)] * 2,
    )(attr8, w1, w2)


def _dense_last(h_in, agg, opa, W1, b1, W2, b2):
    def body(hin_ref, agg_ref, opa_ref, W1r, b1r, W2r, b2r, out_ref):
        z = opa_ref[0, 0] * hin_ref[...] + agg_ref[0] + agg_ref[1]
        t = jnp.maximum(
            jnp.dot(z, W1r[...], preferred_element_type=_f32) + b1r[...], 0.0)
        out_ref[...] = (
            jnp.dot(t, W2r[...], preferred_element_type=_f32) + b2r[...])

    return pl.pallas_call(
        body,
        grid=(NB,),
        in_specs=[
            pl.BlockSpec((R, EMB), lambda i: (i, 0)),
            pl.BlockSpec((2, R, EMB), lambda i: (0, i, 0)),
            pl.BlockSpec((1, 1), lambda i: (0, 0)),
            pl.BlockSpec((EMB, 2 * EMB), lambda i: (0, 0)),
            pl.BlockSpec((1, 2 * EMB), lambda i: (0, 0)),
            pl.BlockSpec((2 * EMB, EMB), lambda i: (0, 0)),
            pl.BlockSpec((1, EMB), lambda i: (0, 0)),
        ],
        out_specs=pl.BlockSpec((R, EMB), lambda i: (i, 0)),
        out_shape=jax.ShapeDtypeStruct((N, EMB), _f32),
    )(h_in, agg, opa, W1, b1, W2, b2)


def _dense_mid(h_in, agg, batch3, vn, opa, W1, b1, W2, b2, vW1, vb1, vW2, vb2,
               const_hin=False):
    def body(hin_ref, agg_ref, b3_ref, vn_ref, opa_ref, W1r, b1r, W2r, b2r,
             vW1r, vb1r, vW2r, vb2r, ho_ref, vno_ref, pooled):
        i = pl.program_id(0)
        if const_hin:
            hin = jnp.broadcast_to(hin_ref[...], (R, EMB))
        else:
            hin = hin_ref[...]
        z = opa_ref[0, 0] * hin + agg_ref[0] + agg_ref[1]
        t = jnp.maximum(
            jnp.dot(z, W1r[...], preferred_element_type=_f32) + b1r[...], 0.0)
        y = jnp.dot(t, W2r[...], preferred_element_type=_f32) + b2r[...]
        ho_ref[...] = jnp.maximum(y, 0.0)

        b = b3_ref[0]
        oh = (lax.broadcasted_iota(jnp.int32, (NG, R), 0) == b).astype(_f32)
        part = jnp.dot(oh, hin, preferred_element_type=_f32)

        @pl.when(i == 0)
        def _():
            pooled[...] = part

        @pl.when(i > 0)
        def _():
            pooled[...] = pooled[...] + part

        @pl.when(i == NB - 1)
        def _():
            vtmp = pooled[...] + vn_ref[...]
            v = jnp.maximum(
                jnp.dot(vtmp, vW1r[...], preferred_element_type=_f32)
                + vb1r[...], 0.0)
            vno_ref[...] = jnp.maximum(
                jnp.dot(v, vW2r[...], preferred_element_type=_f32)
                + vb2r[...], 0.0)

    hin_spec = (pl.BlockSpec((1, EMB), lambda i: (0, 0)) if const_hin
                else pl.BlockSpec((R, EMB), lambda i: (i, 0)))
    return pl.pallas_call(
        body,
        grid=(NB,),
        in_specs=[
            hin_spec,
            pl.BlockSpec((2, R, EMB), lambda i: (0, i, 0)),
            pl.BlockSpec((1, 1, R), lambda i: (i, 0, 0)),
            pl.BlockSpec((NG, EMB), lambda i: (0, 0)),
            pl.BlockSpec((1, 1), lambda i: (0, 0)),
            pl.BlockSpec((EMB, 2 * EMB), lambda i: (0, 0)),
            pl.BlockSpec((1, 2 * EMB), lambda i: (0, 0)),
            pl.BlockSpec((2 * EMB, EMB), lambda i: (0, 0)),
            pl.BlockSpec((1, EMB), lambda i: (0, 0)),
            pl.BlockSpec((EMB, 2 * EMB), lambda i: (0, 0)),
            pl.BlockSpec((1, 2 * EMB), lambda i: (0, 0)),
            pl.BlockSpec((2 * EMB, EMB), lambda i: (0, 0)),
            pl.BlockSpec((1, EMB), lambda i: (0, 0)),
        ],
        out_specs=[
            pl.BlockSpec((R, EMB), lambda i: (i, 0)),
            pl.BlockSpec((NG, EMB), lambda i: (0, 0)),
        ],
        out_shape=[
            jax.ShapeDtypeStruct((N, EMB), _f32),
            jax.ShapeDtypeStruct((NG, EMB), _f32),
        ],
        scratch_shapes=[pltpu.VMEM((NG, EMB), _f32)],
    )(h_in, agg, batch3, vn, opa, W1, b1, W2, b2, vW1, vb1, vW2, vb2)


def _add_vn(h, batch3, vn):
    def body(h_ref, b3_ref, vn_ref, o_ref):
        b = b3_ref[0]
        oh = (lax.broadcasted_iota(jnp.int32, (NG, R), 0) == b).astype(_f32)
        g = lax.dot_general(oh, vn_ref[...], (((0,), (0,)), ((), ())),
                            preferred_element_type=_f32)
        o_ref[...] = h_ref[...] + g

    return pl.pallas_call(
        body,
        grid=(NB,),
        in_specs=[
            pl.BlockSpec((R, EMB), lambda i: (i, 0)),
            pl.BlockSpec((1, 1, R), lambda i: (i, 0, 0)),
            pl.BlockSpec((NG, EMB), lambda i: (0, 0)),
        ],
        out_specs=pl.BlockSpec((R, EMB), lambda i: (i, 0)),
        out_shape=jax.ShapeDtypeStruct((N, EMB), _f32),
    )(h, batch3, vn)


def _sc_mesh():
    return plsc.VectorSubcoreMesh(core_axis_name="c", subcore_axis_name="s")


def _zero_vmem_block(zv, rows):
    def zrow(r, carry):
        for col in range(EMB // 16):
            zv[r, pl.ds(col * 16, 16)] = jnp.zeros((16,), _f32)
        return carry
    lax.fori_loop(0, rows, zrow, 0)


def _zero_agg(zbuf, agg_s, s, rows):
    _zero_vmem_block(zbuf, rows)
    r0 = s * RPT
    for j in range(RPT // rows):
        pltpu.sync_copy(zbuf, agg_s.at[pl.ds(r0 + j * rows, rows)])
    plsc.subcore_barrier()


def _copy_out(agg_s, out_h, c, s, rows):
    plsc.subcore_barrier()
    r0 = s * RPT
    for j in range(RPT // rows):
        r = r0 + j * rows
        pltpu.sync_copy(agg_s.at[pl.ds(r, rows)],
                        out_h.at[pl.ds(c * NP + r, rows)])


def _sc_scatter_only(msg, dstp):
    @functools.partial(
        pl.kernel,
        mesh=_sc_mesh(),
        out_type=jax.ShapeDtypeStruct((NC * NP, EMB), _f32),
        scratch_types=[
            pltpu.VMEM((CHS,), jnp.int32),
            pltpu.VMEM((CHS,), jnp.int32),
            pltpu.VMEM((CHS, EMB), _f32),
            pltpu.VMEM((CHS, EMB), _f32),
            pltpu.VMEM_SHARED((NP, EMB), _f32),
            pltpu.SemaphoreType.DMA,
            pltpu.SemaphoreType.DMA,
        ],
    )
    def k(msg_h, dst_h, out_h, didx0, didx1, buf0, buf1, agg_s, lsem0, lsem1):
        c = lax.axis_index("c")
        s = lax.axis_index("s")
        wid = s * NC + c
        _zero_agg(buf0, agg_s, s, CHS)
        base = wid * WE
        didx = (didx0, didx1)
        buf = (buf0, buf1)
        lsem = (lsem0, lsem1)

        def issue(t, b):
            off = base + t * CHS
            pltpu.async_copy(dst_h.at[pl.ds(off, CHS)], didx[b], lsem[b])
            pltpu.async_copy(msg_h.at[pl.ds(off, CHS)], buf[b], lsem[b])

        def drain(b):
            pltpu.make_async_copy(dst_h.at[pl.ds(0, CHS)], didx[b],
                                  lsem[b]).wait()
            pltpu.make_async_copy(msg_h.at[pl.ds(0, CHS)], buf[b],
                                  lsem[b]).wait()

        issue(0, 0)

        def pair(i, carry):
            t1 = 2 * i + 1
            t2 = 2 * i + 2
            drain(0)
            issue(t1, 1)
            pltpu.sync_copy(buf[0], agg_s.at[didx[0]], add=True)
            drain(1)

            @pl.when(t2 < NCHS)
            def _():
                issue(t2, 0)

            pltpu.sync_copy(buf[1], agg_s.at[didx[1]], add=True)
            return carry

        lax.fori_loop(0, NCHS // 2, pair, 0)
        _copy_out(agg_s, out_h, c, s, CHS)

    return k(msg, dstp).reshape(NC, NP, EMB)


def _sc_edge(ee, srcp, dstp, h_in):
    @functools.partial(
        pl.kernel,
        mesh=_sc_mesh(),
        out_type=jax.ShapeDtypeStruct((NC * NP, EMB), _f32),
        scratch_types=[
            pltpu.VMEM((CHE,), jnp.int32),
            pltpu.VMEM((CHE,), jnp.int32),
            pltpu.VMEM((CHE,), jnp.int32),
            pltpu.VMEM((CHE,), jnp.int32),
            pltpu.VMEM((CHE, EMB), _f32),
            pltpu.VMEM((CHE, EMB), _f32),
            pltpu.VMEM((CHE, EMB), _f32),
            pltpu.VMEM((CHE, EMB), _f32),
            pltpu.VMEM_SHARED((NP, EMB), _f32),
            pltpu.SemaphoreType.DMA,
            pltpu.SemaphoreType.DMA,
            pltpu.SemaphoreType.DMA,
            pltpu.SemaphoreType.DMA,
        ],
    )
    def k(ee_h, src_h, dst_h, hin_h, out_h,
          sidx0, sidx1, didx0, didx1, ebuf0, ebuf1, hbuf0, hbuf1,
          agg_s, lsem0, lsem1, gsem0, gsem1):
        c = lax.axis_index("c")
        s = lax.axis_index("s")
        wid = s * NC + c
        _zero_agg(ebuf0, agg_s, s, CHE)
        base = wid * WE
        sidx = (sidx0, sidx1)
        didx = (didx0, didx1)
        ebuf = (ebuf0, ebuf1)
        hbuf = (hbuf0, hbuf1)
        lsem = (lsem0, lsem1)
        gsem = (gsem0, gsem1)

        def issue_loads(t, b):
            off = base + t * CHE
            pltpu.async_copy(src_h.at[pl.ds(off, CHE)], sidx[b], lsem[b])
            pltpu.async_copy(dst_h.at[pl.ds(off, CHE)], didx[b], lsem[b])
            pltpu.async_copy(ee_h.at[pl.ds(off, CHE)], ebuf[b], lsem[b])

        def drain_loads(b):
            pltpu.make_async_copy(src_h.at[pl.ds(0, CHE)], sidx[b],
                                  lsem[b]).wait()
            pltpu.make_async_copy(dst_h.at[pl.ds(0, CHE)], didx[b],
                                  lsem[b]).wait()
            pltpu.make_async_copy(ee_h.at[pl.ds(0, CHE)], ebuf[b],
                                  lsem[b]).wait()

        def issue_gather(b):
            pltpu.async_copy(hin_h.at[sidx[b]], hbuf[b], gsem[b])

        def drain_gather(b):
            pltpu.make_async_copy(hin_h.at[sidx[b]], hbuf[b],
                                  gsem[b]).wait()

        def compute(b):
            eb = ebuf[b]
            hb = hbuf[b]

            def row(r, cc):
                for u in range(2):
                    rr = r * 2 + u
                    for col in range(EMB // 16):
                        sl = (rr, pl.ds(col * 16, 16))
                        eb[sl] = jnp.maximum(eb[sl] + hb[sl], 0.0)
                return cc

            lax.fori_loop(0, CHE // 2, row, 0)

        def scatter(b):
            pltpu.sync_copy(ebuf[b], agg_s.at[didx[b]], add=True)

        issue_loads(0, 0)
        drain_loads(0)
        issue_gather(0)

        def pair(i, carry):
            t1 = 2 * i + 1
            t2 = 2 * i + 2
            issue_loads(t1, 1)
            drain_gather(0)
            compute(0)
            drain_loads(1)
            issue_gather(1)
            scatter(0)
            @pl.when(t2 < NCHE)
            def _():
                issue_loads(t2, 0)

            drain_gather(1)
            compute(1)

            @pl.when(t2 < NCHE)
            def _():
                drain_loads(0)
                issue_gather(0)

            scatter(1)
            return carry

        lax.fori_loop(0, NCHE // 2, pair, 0)
        _copy_out(agg_s, out_h, c, s, CHS)

    return k(ee, srcp, dstp, h_in).reshape(NC, NP, EMB)


def kernel(params, edge_attr, x, edge_index, batch):
    p = params
    kbn = np.float32(1.0 / np.sqrt(1.0 + BN_EPS))

    c0 = p["node_enc"][0] + p["vn_emb"][0]
    vn0 = jnp.broadcast_to(p["vn_emb"][0], (NG, EMB))

    src = edge_index[0]
    dst = edge_index[1]
    pad = EP - E
    srcp = jnp.concatenate([src, jnp.zeros((pad,), jnp.int32)])
    dstp = jnp.concatenate([dst, jnp.full((pad,), N, jnp.int32)])
    attr8 = jnp.concatenate(
        [edge_attr, jnp.zeros((E, 1), _f32)], axis=1)
    attr8 = jnp.concatenate(
        [attr8, jnp.zeros((pad, 8), _f32)], axis=0)
    attr8 = attr8.at[:E, 7].set(1.0)
    batch3 = batch.reshape(NB, 1, R)

    Ls = []
    for l in range(3):
        q = p["layers"][l]
        g1s = q["g1"] * kbn
        W1 = q["W1"] * g1s[None, :]
        b1 = (q["b1"] * g1s + q["b1n"])[None, :]
        gos = p["bn_g"][l] * kbn
        W2 = q["W2"] * gos[None, :]
        b2 = (q["b2"] * gos + p["bn_b"][l])[None, :]
        bias = q["eeb"] + (c0 if l == 0 else 0.0)
        eeW = jnp.concatenate([q["eeW"], bias[None, :]], axis=0)
        opa = (1.0 + q["eps"]).reshape(1, 1)
        Ls.append((eeW, opa, W1, b1, W2, b2))
    Vs = []
    for l in range(2):
        m = p["vn_mlps"][l]
        gs1 = m["g1"] * kbn
        vW1 = m["W1"] * gs1[None, :]
        vb1 = (m["b1"] * gs1 + m["be1"])[None, :]
        gs2 = m["g2"] * kbn
        vW2 = m["W2"] * gs2[None, :]
        vb2 = (m["b2"] * gs2 + m["be2"])[None, :]
        Vs.append((vW1, vb1, vW2, vb2))

    msg0 = _tc_pre0(attr8, Ls[0][0])
    ee1, ee2 = _tc_pre12(attr8, Ls[1][0], Ls[2][0])

    agg0 = _sc_scatter_only(msg0, dstp)
    h1, vn1 = _dense_mid(c0[None], agg0, batch3, vn0, *Ls[0][1:], *Vs[0],
                         const_hin=True)
    h_in1 = _add_vn(h1, batch3, vn1)

    agg1 = _sc_edge(ee1, srcp, dstp, h_in1)
    h2, vn2 = _dense_mid(h_in1, agg1, batch3, vn1, *Ls[1][1:], *Vs[1])
    h_in2 = _add_vn(h2, batch3, vn2)

    agg2 = _sc_edge(ee2, srcp, dstp, h_in2)
    return _dense_last(h_in2, agg2, *Ls[2][1:])


# speedup vs baseline: 1.0962x; 1.0962x over previous
import functools

import jax
import jax.numpy as jnp
import numpy as np
from jax import lax
from jax.experimental import pallas as pl
from jax.experimental.pallas import tpu as pltpu
from jax.experimental.pallas import tpu_sc as plsc

N = 10000
E = 320000
EMB = 128
NG = 64
BN_EPS = 1e-5

NC = 2
NS = 16
NW = NC * NS
EP = 327680
WE = EP // NW
CHS = 128
NCHS = WE // CHS
CHE = 80
NCHE = WE // CHE
K0 = 174
K1 = 2 * NCHE - K0
NP = 10240
RPT = NP // NS

NB = 5
R = N // NB
BE = 2560

_f32 = jnp.float32


def _tc_pre(attr8, w0, w1, w2):
    def body(a_ref, w0_ref, w1_ref, w2_ref, o0_ref, o1_ref, o2_ref):
        a = a_ref[...]
        o0_ref[...] = jnp.maximum(
            jnp.dot(a, w0_ref[...], preferred_element_type=_f32), 0.0)
        o1_ref[...] = jnp.dot(a, w1_ref[...], preferred_element_type=_f32)
        o2_ref[...] = jnp.dot(a, w2_ref[...], preferred_element_type=_f32)

    wspec = pl.BlockSpec((8, EMB), lambda i: (0, 0))
    return pl.pallas_call(
        body,
        grid=(EP // BE,),
        in_specs=[pl.BlockSpec((BE, 8), lambda i: (i, 0)), wspec, wspec, wspec],
        out_specs=[pl.BlockSpec((BE, EMB), lambda i: (i, 0))] * 3,
        out_shape=[jax.ShapeDtypeStruct((EP, EMB), _f32)] * 3,
    )(attr8, w0, w1, w2)


def _dense_last(h_in, agg, opa, W1, b1, W2, b2):
    def body(hin_ref, agg_ref, opa_ref, W1r, b1r, W2r, b2r, out_ref):
        z = opa_ref[0, 0] * hin_ref[...] + agg_ref[0] + agg_ref[1]
        t = jnp.maximum(
            jnp.dot(z, W1r[...], preferred_element_type=_f32) + b1r[...], 0.0)
        out_ref[...] = (
            jnp.dot(t, W2r[...], preferred_element_type=_f32) + b2r[...])

    return pl.pallas_call(
        body,
        grid=(NB,),
        in_specs=[
            pl.BlockSpec((R, EMB), lambda i: (i, 0)),
            pl.BlockSpec((2, R, EMB), lambda i: (0, i, 0)),
            pl.BlockSpec((1, 1), lambda i: (0, 0)),
            pl.BlockSpec((EMB, 2 * EMB), lambda i: (0, 0)),
            pl.BlockSpec((1, 2 * EMB), lambda i: (0, 0)),
            pl.BlockSpec((2 * EMB, EMB), lambda i: (0, 0)),
            pl.BlockSpec((1, EMB), lambda i: (0, 0)),
        ],
        out_specs=pl.BlockSpec((R, EMB), lambda i: (i, 0)),
        out_shape=jax.ShapeDtypeStruct((N, EMB), _f32),
    )(h_in, agg, opa, W1, b1, W2, b2)


def _dense_mid(h_in, agg, batch3, vn, opa, W1, b1, W2, b2, vW1, vb1, vW2, vb2,
               const_hin=False):
    def body(hin_ref, agg_ref, b3_ref, vn_ref, opa_ref, W1r, b1r, W2r, b2r,
             vW1r, vb1r, vW2r, vb2r, ho_ref, vno_ref, pooled):
        i = pl.program_id(0)
        if const_hin:
            hin = jnp.broadcast_to(hin_ref[...], (R, EMB))
        else:
            hin = hin_ref[...]
        z = opa_ref[0, 0] * hin + agg_ref[0] + agg_ref[1]
        t = jnp.maximum(
            jnp.dot(z, W1r[...], preferred_element_type=_f32) + b1r[...], 0.0)
        y = jnp.dot(t, W2r[...], preferred_element_type=_f32) + b2r[...]
        ho_ref[...] = jnp.maximum(y, 0.0)

        b = b3_ref[0]
        oh = (lax.broadcasted_iota(jnp.int32, (NG, R), 0) == b).astype(_f32)
        part = jnp.dot(oh, hin, preferred_element_type=_f32)

        @pl.when(i == 0)
        def _():
            pooled[...] = part

        @pl.when(i > 0)
        def _():
            pooled[...] = pooled[...] + part

        @pl.when(i == NB - 1)
        def _():
            vtmp = pooled[...] + vn_ref[...]
            v = jnp.maximum(
                jnp.dot(vtmp, vW1r[...], preferred_element_type=_f32)
                + vb1r[...], 0.0)
            vno_ref[...] = jnp.maximum(
                jnp.dot(v, vW2r[...], preferred_element_type=_f32)
                + vb2r[...], 0.0)

    hin_spec = (pl.BlockSpec((1, EMB), lambda i: (0, 0)) if const_hin
                else pl.BlockSpec((R, EMB), lambda i: (i, 0)))
    return pl.pallas_call(
        body,
        grid=(NB,),
        in_specs=[
            hin_spec,
            pl.BlockSpec((2, R, EMB), lambda i: (0, i, 0)),
            pl.BlockSpec((1, 1, R), lambda i: (i, 0, 0)),
            pl.BlockSpec((NG, EMB), lambda i: (0, 0)),
            pl.BlockSpec((1, 1), lambda i: (0, 0)),
            pl.BlockSpec((EMB, 2 * EMB), lambda i: (0, 0)),
            pl.BlockSpec((1, 2 * EMB), lambda i: (0, 0)),
            pl.BlockSpec((2 * EMB, EMB), lambda i: (0, 0)),
            pl.BlockSpec((1, EMB), lambda i: (0, 0)),
            pl.BlockSpec((EMB, 2 * EMB), lambda i: (0, 0)),
            pl.BlockSpec((1, 2 * EMB), lambda i: (0, 0)),
            pl.BlockSpec((2 * EMB, EMB), lambda i: (0, 0)),
            pl.BlockSpec((1, EMB), lambda i: (0, 0)),
        ],
        out_specs=[
            pl.BlockSpec((R, EMB), lambda i: (i, 0)),
            pl.BlockSpec((NG, EMB), lambda i: (0, 0)),
        ],
        out_shape=[
            jax.ShapeDtypeStruct((N, EMB), _f32),
            jax.ShapeDtypeStruct((NG, EMB), _f32),
        ],
        scratch_shapes=[pltpu.VMEM((NG, EMB), _f32)],
    )(h_in, agg, batch3, vn, opa, W1, b1, W2, b2, vW1, vb1, vW2, vb2)


def _add_vn(h, batch3, vn):
    def body(h_ref, b3_ref, vn_ref, o_ref):
        b = b3_ref[0]
        oh = (lax.broadcasted_iota(jnp.int32, (NG, R), 0) == b).astype(_f32)
        g = lax.dot_general(oh, vn_ref[...], (((0,), (0,)), ((), ())),
                            preferred_element_type=_f32)
        o_ref[...] = h_ref[...] + g

    return pl.pallas_call(
        body,
        grid=(NB,),
        in_specs=[
            pl.BlockSpec((R, EMB), lambda i: (i, 0)),
            pl.BlockSpec((1, 1, R), lambda i: (i, 0, 0)),
            pl.BlockSpec((NG, EMB), lambda i: (0, 0)),
        ],
        out_specs=pl.BlockSpec((R, EMB), lambda i: (i, 0)),
        out_shape=jax.ShapeDtypeStruct((N, EMB), _f32),
    )(h, batch3, vn)


def _sc_mesh():
    return plsc.VectorSubcoreMesh(core_axis_name="c", subcore_axis_name="s")


def _zero_vmem_block(zv, rows):
    def zrow(r, carry):
        for col in range(EMB // 16):
            zv[r, pl.ds(col * 16, 16)] = jnp.zeros((16,), _f32)
        return carry
    lax.fori_loop(0, rows, zrow, 0)


def _zero_agg(zbuf, agg_s, s, rows):
    _zero_vmem_block(zbuf, rows)
    r0 = s * RPT
    for j in range(RPT // rows):
        pltpu.sync_copy(zbuf, agg_s.at[pl.ds(r0 + j * rows, rows)])
    plsc.subcore_barrier()


def _copy_out(agg_s, out_h, c, s, rows):
    plsc.subcore_barrier()
    r0 = s * RPT
    for j in range(RPT // rows):
        r = r0 + j * rows
        pltpu.sync_copy(agg_s.at[pl.ds(r, rows)],
                        out_h.at[pl.ds(c * NP + r, rows)])


def _sc_scatter_only(msg, dstp):
    @functools.partial(
        pl.kernel,
        mesh=_sc_mesh(),
        out_type=jax.ShapeDtypeStruct((NC * NP, EMB), _f32),
        scratch_types=[
            pltpu.VMEM((CHS,), jnp.int32),
            pltpu.VMEM((CHS,), jnp.int32),
            pltpu.VMEM((CHS, EMB), _f32),
            pltpu.VMEM((CHS, EMB), _f32),
            pltpu.VMEM_SHARED((NP, EMB), _f32),
            pltpu.SemaphoreType.DMA,
            pltpu.SemaphoreType.DMA,
        ],
    )
    def k(msg_h, dst_h, out_h, didx0, didx1, buf0, buf1, agg_s, lsem0, lsem1):
        c = lax.axis_index("c")
        s = lax.axis_index("s")
        wid = s * NC + c
        _zero_agg(buf0, agg_s, s, CHS)
        base = wid * WE
        didx = (didx0, didx1)
        buf = (buf0, buf1)
        lsem = (lsem0, lsem1)

        def issue(t, b):
            off = base + t * CHS
            pltpu.async_copy(dst_h.at[pl.ds(off, CHS)], didx[b], lsem[b])
            pltpu.async_copy(msg_h.at[pl.ds(off, CHS)], buf[b], lsem[b])

        def drain(b):
            pltpu.make_async_copy(dst_h.at[pl.ds(0, CHS)], didx[b],
                                  lsem[b]).wait()
            pltpu.make_async_copy(msg_h.at[pl.ds(0, CHS)], buf[b],
                                  lsem[b]).wait()

        issue(0, 0)

        def pair(i, carry):
            t1 = 2 * i + 1
            t2 = 2 * i + 2
            drain(0)
            issue(t1, 1)
            pltpu.sync_copy(buf[0], agg_s.at[didx[0]], add=True)
            drain(1)

            @pl.when(t2 < NCHS)
            def _():
                issue(t2, 0)

            pltpu.sync_copy(buf[1], agg_s.at[didx[1]], add=True)
            return carry

        lax.fori_loop(0, NCHS // 2, pair, 0)
        _copy_out(agg_s, out_h, c, s, CHS)

    return k(msg, dstp).reshape(NC, NP, EMB)


def _sc_edge(ee, srcp, dstp, h_in):
    @functools.partial(
        pl.kernel,
        mesh=_sc_mesh(),
        out_type=jax.ShapeDtypeStruct((NC * NP, EMB), _f32),
        scratch_types=[
            pltpu.VMEM((CHE,), jnp.int32),
            pltpu.VMEM((CHE,), jnp.int32),
            pltpu.VMEM((CHE,), jnp.int32),
            pltpu.VMEM((CHE,), jnp.int32),
            pltpu.VMEM((CHE, EMB), _f32),
            pltpu.VMEM((CHE, EMB), _f32),
            pltpu.VMEM((CHE, EMB), _f32),
            pltpu.VMEM((CHE, EMB), _f32),
            pltpu.VMEM_SHARED((NP, EMB), _f32),
            pltpu.SemaphoreType.DMA,
            pltpu.SemaphoreType.DMA,
            pltpu.SemaphoreType.DMA,
            pltpu.SemaphoreType.DMA,
        ],
    )
    def k(ee_h, src_h, dst_h, hin_h, out_h,
          sidx0, sidx1, didx0, didx1, ebuf0, ebuf1, hbuf0, hbuf1,
          agg_s, lsem0, lsem1, gsem0, gsem1):
        c = lax.axis_index("c")
        s = lax.axis_index("s")
        _zero_agg(ebuf0, agg_s, s, CHE)
        sidx = (sidx0, sidx1)
        didx = (didx0, didx1)
        ebuf = (ebuf0, ebuf1)
        hbuf = (hbuf0, hbuf1)
        lsem = (lsem0, lsem1)
        gsem = (gsem0, gsem1)

        def issue_loads(base, t, b):
            off = base + t * CHE
            pltpu.async_copy(src_h.at[pl.ds(off, CHE)], sidx[b], lsem[b])
            pltpu.async_copy(dst_h.at[pl.ds(off, CHE)], didx[b], lsem[b])
            pltpu.async_copy(ee_h.at[pl.ds(off, CHE)], ebuf[b], lsem[b])

        def drain_loads(b):
            pltpu.make_async_copy(src_h.at[pl.ds(0, CHE)], sidx[b],
                                  lsem[b]).wait()
            pltpu.make_async_copy(dst_h.at[pl.ds(0, CHE)], didx[b],
                                  lsem[b]).wait()
            pltpu.make_async_copy(ee_h.at[pl.ds(0, CHE)], ebuf[b],
                                  lsem[b]).wait()

        def issue_gather(b):
            pltpu.async_copy(hin_h.at[sidx[b]], hbuf[b], gsem[b])

        def drain_gather(b):
            pltpu.make_async_copy(hin_h.at[sidx[b]], hbuf[b],
                                  gsem[b]).wait()

        def compute(b):
            eb = ebuf[b]
            hb = hbuf[b]

            def row(r, cc):
                for u in range(2):
                    rr = r * 2 + u
                    for col in range(EMB // 16):
                        sl = (rr, pl.ds(col * 16, 16))
                        eb[sl] = jnp.maximum(eb[sl] + hb[sl], 0.0)
                return cc

            lax.fori_loop(0, CHE // 2, row, 0)

        def scatter(b):
            pltpu.sync_copy(ebuf[b], agg_s.at[didx[b]], add=True)

        def run(base, nch):
            issue_loads(base, 0, 0)
            drain_loads(0)
            issue_gather(0)

            def pair(i, carry):
                t1 = 2 * i + 1
                t2 = 2 * i + 2
                issue_loads(base, t1, 1)
                drain_gather(0)
                compute(0)
                drain_loads(1)
                issue_gather(1)
                scatter(0)
                @pl.when(t2 < nch)
                def _():
                    issue_loads(base, t2, 0)

                drain_gather(1)
                compute(1)

                @pl.when(t2 < nch)
                def _():
                    drain_loads(0)
                    issue_gather(0)

                scatter(1)
                return carry

            lax.fori_loop(0, nch // 2, pair, 0)

        @pl.when(c == 0)
        def _():
            run(s * K0 * CHE, K0)

        @pl.when(c == 1)
        def _():
            run((NS * K0 + s * K1) * CHE, K1)

        _copy_out(agg_s, out_h, c, s, CHS)

    return k(ee, srcp, dstp, h_in).reshape(NC, NP, EMB)


def kernel(params, edge_attr, x, edge_index, batch):
    p = params
    kbn = np.float32(1.0 / np.sqrt(1.0 + BN_EPS))

    c0 = p["node_enc"][0] + p["vn_emb"][0]
    vn0 = jnp.broadcast_to(p["vn_emb"][0], (NG, EMB))

    src = edge_index[0]
    dst = edge_index[1]
    pad = EP - E
    srcp = jnp.concatenate([src, jnp.zeros((pad,), jnp.int32)])
    dstp = jnp.concatenate([dst, jnp.full((pad,), N, jnp.int32)])
    attr8 = jnp.concatenate(
        [edge_attr, jnp.zeros((E, 1), _f32)], axis=1)
    attr8 = jnp.concatenate(
        [attr8, jnp.zeros((pad, 8), _f32)], axis=0)
    attr8 = attr8.at[:E, 7].set(1.0)
    batch3 = batch.reshape(NB, 1, R)

    Ls = []
    for l in range(3):
        q = p["layers"][l]
        g1s = q["g1"] * kbn
        W1 = q["W1"] * g1s[None, :]
        b1 = (q["b1"] * g1s + q["b1n"])[None, :]
        gos = p["bn_g"][l] * kbn
        W2 = q["W2"] * gos[None, :]
        b2 = (q["b2"] * gos + p["bn_b"][l])[None, :]
        bias = q["eeb"] + (c0 if l == 0 else 0.0)
        eeW = jnp.concatenate([q["eeW"], bias[None, :]], axis=0)
        opa = (1.0 + q["eps"]).reshape(1, 1)
        Ls.append((eeW, opa, W1, b1, W2, b2))
    Vs = []
    for l in range(2):
        m = p["vn_mlps"][l]
        gs1 = m["g1"] * kbn
        vW1 = m["W1"] * gs1[None, :]
        vb1 = (m["b1"] * gs1 + m["be1"])[None, :]
        gs2 = m["g2"] * kbn
        vW2 = m["W2"] * gs2[None, :]
        vb2 = (m["b2"] * gs2 + m["be2"])[None, :]
        Vs.append((vW1, vb1, vW2, vb2))

    msg0, ee1, ee2 = _tc_pre(attr8, Ls[0][0], Ls[1][0], Ls[2][0])

    agg0 = _sc_scatter_only(msg0, dstp)
    h1, vn1 = _dense_mid(c0[None], agg0, batch3, vn0, *Ls[0][1:], *Vs[0],
                         const_hin=True)
    h_in1 = _add_vn(h1, batch3, vn1)

    agg1 = _sc_edge(ee1, srcp, dstp, h_in1)
    h2, vn2 = _dense_mid(h_in1, agg1, batch3, vn1, *Ls[1][1:], *Vs[1])
    h_in2 = _add_vn(h2, batch3, vn2)

    agg2 = _sc_edge(ee2, srcp, dstp, h_in2)
    return _dense_last(h_in2, agg2, *Ls[2][1:])


# speedup vs baseline: 1.1459x; 1.0453x over previous
import functools

import jax
import jax.numpy as jnp
import numpy as np
from jax import lax
from jax.experimental import pallas as pl
from jax.experimental.pallas import tpu as pltpu
from jax.experimental.pallas import tpu_sc as plsc

N = 10000
E = 320000
EMB = 128
NG = 64
BN_EPS = 1e-5

NC = 2
NS = 16
NW = NC * NS
EP = 327680
WE = EP // NW
CHS = 128
NCHS = WE // CHS
CHE = 80
NCHE = WE // CHE
K0 = 174
K1 = 2 * NCHE - K0
NP = 10240
RPT = NP // NS

NB = 5
R = N // NB
BE = 2560

_f32 = jnp.float32


def _tc_pre(attr8, w0, w1, w2):
    def body(a_ref, w0_ref, w1_ref, w2_ref, o0_ref, o1_ref, o2_ref):
        a = a_ref[...]
        o0_ref[...] = jnp.maximum(
            jnp.dot(a, w0_ref[...], preferred_element_type=_f32), 0.0)
        o1_ref[...] = jnp.dot(a, w1_ref[...], preferred_element_type=_f32)
        o2_ref[...] = jnp.dot(a, w2_ref[...], preferred_element_type=_f32)

    wspec = pl.BlockSpec((8, EMB), lambda i: (0, 0))
    return pl.pallas_call(
        body,
        grid=(EP // BE,),
        in_specs=[pl.BlockSpec((BE, 8), lambda i: (i, 0)), wspec, wspec, wspec],
        out_specs=[pl.BlockSpec((BE, EMB), lambda i: (i, 0))] * 3,
        out_shape=[jax.ShapeDtypeStruct((EP, EMB), _f32)] * 3,
    )(attr8, w0, w1, w2)


def _dense_last(h_in, agg, opa, W1, b1, W2, b2):
    def body(hin_ref, agg_ref, opa_ref, W1r, b1r, W2r, b2r, out_ref):
        z = opa_ref[0, 0] * hin_ref[...] + agg_ref[0] + agg_ref[1]
        t = jnp.maximum(
            jnp.dot(z, W1r[...], preferred_element_type=_f32) + b1r[...], 0.0)
        out_ref[...] = (
            jnp.dot(t, W2r[...], preferred_element_type=_f32) + b2r[...])

    return pl.pallas_call(
        body,
        grid=(NB,),
        in_specs=[
            pl.BlockSpec((R, EMB), lambda i: (i, 0)),
            pl.BlockSpec((2, R, EMB), lambda i: (0, i, 0)),
            pl.BlockSpec((1, 1), lambda i: (0, 0)),
            pl.BlockSpec((EMB, 2 * EMB), lambda i: (0, 0)),
            pl.BlockSpec((1, 2 * EMB), lambda i: (0, 0)),
            pl.BlockSpec((2 * EMB, EMB), lambda i: (0, 0)),
            pl.BlockSpec((1, EMB), lambda i: (0, 0)),
        ],
        out_specs=pl.BlockSpec((R, EMB), lambda i: (i, 0)),
        out_shape=jax.ShapeDtypeStruct((N, EMB), _f32),
    )(h_in, agg, opa, W1, b1, W2, b2)


def _dense_mid(h_in, agg, batch3, vn, opa, W1, b1, W2, b2, vW1, vb1, vW2, vb2,
               const_hin=False):
    def body(hin_ref, agg_ref, b3_ref, vn_ref, opa_ref, W1r, b1r, W2r, b2r,
             vW1r, vb1r, vW2r, vb2r, ho_ref, vno_ref, pooled, hscr):
        i = pl.program_id(0)
        b = b3_ref[0]
        oh = (lax.broadcasted_iota(jnp.int32, (NG, R), 0) == b).astype(_f32)

        @pl.when(i < NB)
        def _():
            if const_hin:
                hin = jnp.broadcast_to(hin_ref[...], (R, EMB))
            else:
                hin = hin_ref[...]
            z = opa_ref[0, 0] * hin + agg_ref[0] + agg_ref[1]
            t = jnp.maximum(
                jnp.dot(z, W1r[...], preferred_element_type=_f32) + b1r[...],
                0.0)
            y = jnp.dot(t, W2r[...], preferred_element_type=_f32) + b2r[...]
            hscr[i] = jnp.maximum(y, 0.0)
            part = jnp.dot(oh, hin, preferred_element_type=_f32)

            @pl.when(i == 0)
            def _():
                pooled[...] = part

            @pl.when(i > 0)
            def _():
                pooled[...] = pooled[...] + part

            @pl.when(i == NB - 1)
            def _():
                vtmp = pooled[...] + vn_ref[...]
                v = jnp.maximum(
                    jnp.dot(vtmp, vW1r[...], preferred_element_type=_f32)
                    + vb1r[...], 0.0)
                vno_ref[...] = jnp.maximum(
                    jnp.dot(v, vW2r[...], preferred_element_type=_f32)
                    + vb2r[...], 0.0)

        @pl.when(i >= NB)
        def _():
            g = lax.dot_general(oh, vno_ref[...], (((0,), (0,)), ((), ())),
                                preferred_element_type=_f32)
            ho_ref[...] = hscr[i - NB] + g

    hin_spec = (pl.BlockSpec((1, EMB), lambda i: (0, 0)) if const_hin
                else pl.BlockSpec((R, EMB), lambda i: (jnp.minimum(i, NB - 1),
                                                       0)))
    wrap = lambda i: jnp.where(i < NB, i, i - NB)
    return pl.pallas_call(
        body,
        grid=(2 * NB,),
        in_specs=[
            hin_spec,
            pl.BlockSpec((2, R, EMB), lambda i: (0, jnp.minimum(i, NB - 1),
                                                 0)),
            pl.BlockSpec((1, 1, R), lambda i: (wrap(i), 0, 0)),
            pl.BlockSpec((NG, EMB), lambda i: (0, 0)),
            pl.BlockSpec((1, 1), lambda i: (0, 0)),
            pl.BlockSpec((EMB, 2 * EMB), lambda i: (0, 0)),
            pl.BlockSpec((1, 2 * EMB), lambda i: (0, 0)),
            pl.BlockSpec((2 * EMB, EMB), lambda i: (0, 0)),
            pl.BlockSpec((1, EMB), lambda i: (0, 0)),
            pl.BlockSpec((EMB, 2 * EMB), lambda i: (0, 0)),
            pl.BlockSpec((1, 2 * EMB), lambda i: (0, 0)),
            pl.BlockSpec((2 * EMB, EMB), lambda i: (0, 0)),
            pl.BlockSpec((1, EMB), lambda i: (0, 0)),
        ],
        out_specs=[
            pl.BlockSpec((R, EMB), lambda i: (jnp.where(i < NB, 0, i - NB),
                                              0)),
            pl.BlockSpec((NG, EMB), lambda i: (0, 0)),
        ],
        out_shape=[
            jax.ShapeDtypeStruct((N, EMB), _f32),
            jax.ShapeDtypeStruct((NG, EMB), _f32),
        ],
        scratch_shapes=[pltpu.VMEM((NG, EMB), _f32),
                        pltpu.VMEM((NB, R, EMB), _f32)],
    )(h_in, agg, batch3, vn, opa, W1, b1, W2, b2, vW1, vb1, vW2, vb2)


def _sc_mesh():
    return plsc.VectorSubcoreMesh(core_axis_name="c", subcore_axis_name="s")


def _zero_vmem_block(zv, rows):
    def zrow(r, carry):
        for col in range(EMB // 16):
            zv[r, pl.ds(col * 16, 16)] = jnp.zeros((16,), _f32)
        return carry
    lax.fori_loop(0, rows, zrow, 0)


def _zero_agg(zbuf, agg_s, s, rows):
    _zero_vmem_block(zbuf, rows)
    r0 = s * RPT
    for j in range(RPT // rows):
        pltpu.sync_copy(zbuf, agg_s.at[pl.ds(r0 + j * rows, rows)])
    plsc.subcore_barrier()


def _copy_out(agg_s, out_h, c, s, rows):
    plsc.subcore_barrier()
    r0 = s * RPT
    for j in range(RPT // rows):
        r = r0 + j * rows
        pltpu.sync_copy(agg_s.at[pl.ds(r, rows)],
                        out_h.at[pl.ds(c * NP + r, rows)])


def _sc_scatter_only(msg, dstp):
    @functools.partial(
        pl.kernel,
        mesh=_sc_mesh(),
        out_type=jax.ShapeDtypeStruct((NC * NP, EMB), _f32),
        scratch_types=[
            pltpu.VMEM((CHS,), jnp.int32),
            pltpu.VMEM((CHS,), jnp.int32),
            pltpu.VMEM((CHS, EMB), _f32),
            pltpu.VMEM((CHS, EMB), _f32),
            pltpu.VMEM_SHARED((NP, EMB), _f32),
            pltpu.SemaphoreType.DMA,
            pltpu.SemaphoreType.DMA,
        ],
    )
    def k(msg_h, dst_h, out_h, didx0, didx1, buf0, buf1, agg_s, lsem0, lsem1):
        c = lax.axis_index("c")
        s = lax.axis_index("s")
        wid = s * NC + c
        _zero_agg(buf0, agg_s, s, CHS)
        base = wid * WE
        didx = (didx0, didx1)
        buf = (buf0, buf1)
        lsem = (lsem0, lsem1)

        def issue(t, b):
            off = base + t * CHS
            pltpu.async_copy(dst_h.at[pl.ds(off, CHS)], didx[b], lsem[b])
            pltpu.async_copy(msg_h.at[pl.ds(off, CHS)], buf[b], lsem[b])

        def drain(b):
            pltpu.make_async_copy(dst_h.at[pl.ds(0, CHS)], didx[b],
                                  lsem[b]).wait()
            pltpu.make_async_copy(msg_h.at[pl.ds(0, CHS)], buf[b],
                                  lsem[b]).wait()

        issue(0, 0)

        def pair(i, carry):
            t1 = 2 * i + 1
            t2 = 2 * i + 2
            drain(0)
            issue(t1, 1)
            pltpu.sync_copy(buf[0], agg_s.at[didx[0]], add=True)
            drain(1)

            @pl.when(t2 < NCHS)
            def _():
                issue(t2, 0)

            pltpu.sync_copy(buf[1], agg_s.at[didx[1]], add=True)
            return carry

        lax.fori_loop(0, NCHS // 2, pair, 0)
        _copy_out(agg_s, out_h, c, s, CHS)

    return k(msg, dstp).reshape(NC, NP, EMB)


def _sc_edge(ee, srcp, dstp, h_in):
    @functools.partial(
        pl.kernel,
        mesh=_sc_mesh(),
        out_type=jax.ShapeDtypeStruct((NC * NP, EMB), _f32),
        scratch_types=[
            pltpu.VMEM((CHE,), jnp.int32),
            pltpu.VMEM((CHE,), jnp.int32),
            pltpu.VMEM((CHE,), jnp.int32),
            pltpu.VMEM((CHE,), jnp.int32),
            pltpu.VMEM((CHE, EMB), _f32),
            pltpu.VMEM((CHE, EMB), _f32),
            pltpu.VMEM((CHE, EMB), _f32),
            pltpu.VMEM((CHE, EMB), _f32),
            pltpu.VMEM_SHARED((NP, EMB), _f32),
            pltpu.SemaphoreType.DMA,
            pltpu.SemaphoreType.DMA,
            pltpu.SemaphoreType.DMA,
            pltpu.SemaphoreType.DMA,
        ],
    )
    def k(ee_h, src_h, dst_h, hin_h, out_h,
          sidx0, sidx1, didx0, didx1, ebuf0, ebuf1, hbuf0, hbuf1,
          agg_s, lsem0, lsem1, gsem0, gsem1):
        c = lax.axis_index("c")
        s = lax.axis_index("s")
        _zero_agg(ebuf0, agg_s, s, CHE)
        sidx = (sidx0, sidx1)
        didx = (didx0, didx1)
        ebuf = (ebuf0, ebuf1)
        hbuf = (hbuf0, hbuf1)
        lsem = (lsem0, lsem1)
        gsem = (gsem0, gsem1)

        def issue_loads(base, t, b):
            off = base + t * CHE
            pltpu.async_copy(src_h.at[pl.ds(off, CHE)], sidx[b], lsem[b])
            pltpu.async_copy(dst_h.at[pl.ds(off, CHE)], didx[b], lsem[b])
            pltpu.async_copy(ee_h.at[pl.ds(off, CHE)], ebuf[b], lsem[b])

        def drain_loads(b):
            pltpu.make_async_copy(src_h.at[pl.ds(0, CHE)], sidx[b],
                                  lsem[b]).wait()
            pltpu.make_async_copy(dst_h.at[pl.ds(0, CHE)], didx[b],
                                  lsem[b]).wait()
            pltpu.make_async_copy(ee_h.at[pl.ds(0, CHE)], ebuf[b],
                                  lsem[b]).wait()

        def issue_gather(b):
            pltpu.async_copy(hin_h.at[sidx[b]], hbuf[b], gsem[b])

        def drain_gather(b):
            pltpu.make_async_copy(hin_h.at[sidx[b]], hbuf[b],
                                  gsem[b]).wait()

        def compute(b):
            eb = ebuf[b]
            hb = hbuf[b]

            def row(r, cc):
                for u in range(2):
                    rr = r * 2 + u
                    for col in range(EMB // 16):
                        sl = (rr, pl.ds(col * 16, 16))
                        eb[sl] = jnp.maximum(eb[sl] + hb[sl], 0.0)
                return cc

            lax.fori_loop(0, CHE // 2, row, 0)

        def scatter(b):
            pltpu.sync_copy(ebuf[b], agg_s.at[didx[b]], add=True)

        def run(base, nch):
            issue_loads(base, 0, 0)
            drain_loads(0)
            issue_gather(0)

            def pair(i, carry):
                t1 = 2 * i + 1
                t2 = 2 * i + 2
                issue_loads(base, t1, 1)
                drain_gather(0)
                compute(0)
                drain_loads(1)
                issue_gather(1)
                scatter(0)
                @pl.when(t2 < nch)
                def _():
                    issue_loads(base, t2, 0)

                drain_gather(1)
                compute(1)

                @pl.when(t2 < nch)
                def _():
                    drain_loads(0)
                    issue_gather(0)

                scatter(1)
                return carry

            lax.fori_loop(0, nch // 2, pair, 0)

        @pl.when(c == 0)
        def _():
            run(s * K0 * CHE, K0)

        @pl.when(c == 1)
        def _():
            run((NS * K0 + s * K1) * CHE, K1)

        _copy_out(agg_s, out_h, c, s, CHS)

    return k(ee, srcp, dstp, h_in).reshape(NC, NP, EMB)


def kernel(params, edge_attr, x, edge_index, batch):
    p = params
    kbn = np.float32(1.0 / np.sqrt(1.0 + BN_EPS))

    c0 = p["node_enc"][0] + p["vn_emb"][0]
    vn0 = jnp.broadcast_to(p["vn_emb"][0], (NG, EMB))

    src = edge_index[0]
    dst = edge_index[1]
    pad = EP - E
    srcp = jnp.concatenate([src, jnp.zeros((pad,), jnp.int32)])
    dstp = jnp.concatenate([dst, jnp.full((pad,), N, jnp.int32)])
    attr8 = jnp.concatenate(
        [edge_attr, jnp.zeros((E, 1), _f32)], axis=1)
    attr8 = jnp.concatenate(
        [attr8, jnp.zeros((pad, 8), _f32)], axis=0)
    attr8 = attr8.at[:E, 7].set(1.0)
    batch3 = batch.reshape(NB, 1, R)

    Ls = []
    for l in range(3):
        q = p["layers"][l]
        g1s = q["g1"] * kbn
        W1 = q["W1"] * g1s[None, :]
        b1 = (q["b1"] * g1s + q["b1n"])[None, :]
        gos = p["bn_g"][l] * kbn
        W2 = q["W2"] * gos[None, :]
        b2 = (q["b2"] * gos + p["bn_b"][l])[None, :]
        bias = q["eeb"] + (c0 if l == 0 else 0.0)
        eeW = jnp.concatenate([q["eeW"], bias[None, :]], axis=0)
        opa = (1.0 + q["eps"]).reshape(1, 1)
        Ls.append((eeW, opa, W1, b1, W2, b2))
    Vs = []
    for l in range(2):
        m = p["vn_mlps"][l]
        gs1 = m["g1"] * kbn
        vW1 = m["W1"] * gs1[None, :]
        vb1 = (m["b1"] * gs1 + m["be1"])[None, :]
        gs2 = m["g2"] * kbn
        vW2 = m["W2"] * gs2[None, :]
        vb2 = (m["b2"] * gs2 + m["be2"])[None, :]
        Vs.append((vW1, vb1, vW2, vb2))

    msg0, ee1, ee2 = _tc_pre(attr8, Ls[0][0], Ls[1][0], Ls[2][0])

    agg0 = _sc_scatter_only(msg0, dstp)
    h_in1, vn1 = _dense_mid(c0[None], agg0, batch3, vn0, *Ls[0][1:], *Vs[0],
                            const_hin=True)

    agg1 = _sc_edge(ee1, srcp, dstp, h_in1)
    h_in2, vn2 = _dense_mid(h_in1, agg1, batch3, vn1, *Ls[1][1:], *Vs[1])

    agg2 = _sc_edge(ee2, srcp, dstp, h_in2)
    return _dense_last(h_in2, agg2, *Ls[2][1:])


# speedup vs baseline: 1.2001x; 1.0473x over previous
import functools

import jax
import jax.numpy as jnp
import numpy as np
from jax import lax
from jax.experimental import pallas as pl
from jax.experimental.pallas import tpu as pltpu
from jax.experimental.pallas import tpu_sc as plsc

N = 10000
E = 320000
EMB = 128
NG = 64
BN_EPS = 1e-5

NC = 2
NS = 16
NW = NC * NS
EP = 327680
WE = EP // NW
CHS = 128
NCHS = WE // CHS
CHE = 80
NCHE = WE // CHE
K0 = 202
K1 = 2 * NCHE - K0
NP = 10240
RPT = NP // NS

NB = 5
R = N // NB
BE = 2560

_f32 = jnp.float32


def _tc_pre(attr8, w0, w1, w2):
    def body(a_ref, w0_ref, w1_ref, w2_ref, o0_ref, o1_ref, o2_ref):
        a = a_ref[...]
        o0_ref[...] = jnp.maximum(
            jnp.dot(a, w0_ref[...], preferred_element_type=_f32), 0.0)
        o1_ref[...] = jnp.dot(a, w1_ref[...], preferred_element_type=_f32)
        o2_ref[...] = jnp.dot(a, w2_ref[...], preferred_element_type=_f32)

    wspec = pl.BlockSpec((8, EMB), lambda i: (0, 0))
    return pl.pallas_call(
        body,
        grid=(EP // BE,),
        in_specs=[pl.BlockSpec((BE, 8), lambda i: (i, 0)), wspec, wspec, wspec],
        out_specs=[pl.BlockSpec((BE, EMB), lambda i: (i, 0))] * 3,
        out_shape=[jax.ShapeDtypeStruct((EP, EMB), _f32)] * 3,
    )(attr8, w0, w1, w2)


def _dense_last(h_in, agg, opa, W1, b1, W2, b2):
    def body(hin_ref, agg_ref, opa_ref, W1r, b1r, W2r, b2r, out_ref):
        z = opa_ref[0, 0] * hin_ref[...] + agg_ref[0] + agg_ref[1]
        t = jnp.maximum(
            jnp.dot(z, W1r[...], preferred_element_type=_f32) + b1r[...], 0.0)
        out_ref[...] = (
            jnp.dot(t, W2r[...], preferred_element_type=_f32) + b2r[...])

    return pl.pallas_call(
        body,
        grid=(NB,),
        in_specs=[
            pl.BlockSpec((R, EMB), lambda i: (i, 0)),
            pl.BlockSpec((2, R, EMB), lambda i: (0, i, 0)),
            pl.BlockSpec((1, 1), lambda i: (0, 0)),
            pl.BlockSpec((EMB, 2 * EMB), lambda i: (0, 0)),
            pl.BlockSpec((1, 2 * EMB), lambda i: (0, 0)),
            pl.BlockSpec((2 * EMB, EMB), lambda i: (0, 0)),
            pl.BlockSpec((1, EMB), lambda i: (0, 0)),
        ],
        out_specs=pl.BlockSpec((R, EMB), lambda i: (i, 0)),
        out_shape=jax.ShapeDtypeStruct((N, EMB), _f32),
    )(h_in, agg, opa, W1, b1, W2, b2)


def _dense_mid(h_in, agg, batch3, vn, opa, W1, b1, W2, b2, vW1, vb1, vW2, vb2,
               const_hin=False):
    def body(hin_ref, agg_ref, b3_ref, vn_ref, opa_ref, W1r, b1r, W2r, b2r,
             vW1r, vb1r, vW2r, vb2r, ho_ref, vno_ref, pooled, hscr):
        i = pl.program_id(0)
        b = b3_ref[0]
        oh = (lax.broadcasted_iota(jnp.int32, (NG, R), 0) == b).astype(_f32)

        @pl.when(i < NB)
        def _():
            if const_hin:
                hin = jnp.broadcast_to(hin_ref[...], (R, EMB))
            else:
                hin = hin_ref[...]
            z = opa_ref[0, 0] * hin + agg_ref[0] + agg_ref[1]
            t = jnp.maximum(
                jnp.dot(z, W1r[...], preferred_element_type=_f32) + b1r[...],
                0.0)
            y = jnp.dot(t, W2r[...], preferred_element_type=_f32) + b2r[...]
            hscr[i] = jnp.maximum(y, 0.0)
            part = jnp.dot(oh, hin, preferred_element_type=_f32)

            @pl.when(i == 0)
            def _():
                pooled[...] = part

            @pl.when(i > 0)
            def _():
                pooled[...] = pooled[...] + part

            @pl.when(i == NB - 1)
            def _():
                vtmp = pooled[...] + vn_ref[...]
                v = jnp.maximum(
                    jnp.dot(vtmp, vW1r[...], preferred_element_type=_f32)
                    + vb1r[...], 0.0)
                vno_ref[...] = jnp.maximum(
                    jnp.dot(v, vW2r[...], preferred_element_type=_f32)
                    + vb2r[...], 0.0)

        @pl.when(i >= NB)
        def _():
            g = lax.dot_general(oh, vno_ref[...], (((0,), (0,)), ((), ())),
                                preferred_element_type=_f32)
            ho_ref[...] = hscr[i - NB] + g

    hin_spec = (pl.BlockSpec((1, EMB), lambda i: (0, 0)) if const_hin
                else pl.BlockSpec((R, EMB), lambda i: (jnp.minimum(i, NB - 1),
                                                       0)))
    wrap = lambda i: jnp.where(i < NB, i, i - NB)
    return pl.pallas_call(
        body,
        grid=(2 * NB,),
        in_specs=[
            hin_spec,
            pl.BlockSpec((2, R, EMB), lambda i: (0, jnp.minimum(i, NB - 1),
                                                 0)),
            pl.BlockSpec((1, 1, R), lambda i: (wrap(i), 0, 0)),
            pl.BlockSpec((NG, EMB), lambda i: (0, 0)),
            pl.BlockSpec((1, 1), lambda i: (0, 0)),
            pl.BlockSpec((EMB, 2 * EMB), lambda i: (0, 0)),
            pl.BlockSpec((1, 2 * EMB), lambda i: (0, 0)),
            pl.BlockSpec((2 * EMB, EMB), lambda i: (0, 0)),
            pl.BlockSpec((1, EMB), lambda i: (0, 0)),
            pl.BlockSpec((EMB, 2 * EMB), lambda i: (0, 0)),
            pl.BlockSpec((1, 2 * EMB), lambda i: (0, 0)),
            pl.BlockSpec((2 * EMB, EMB), lambda i: (0, 0)),
            pl.BlockSpec((1, EMB), lambda i: (0, 0)),
        ],
        out_specs=[
            pl.BlockSpec((R, EMB), lambda i: (jnp.where(i < NB, 0, i - NB),
                                              0)),
            pl.BlockSpec((NG, EMB), lambda i: (0, 0)),
        ],
        out_shape=[
            jax.ShapeDtypeStruct((N, EMB), _f32),
            jax.ShapeDtypeStruct((NG, EMB), _f32),
        ],
        scratch_shapes=[pltpu.VMEM((NG, EMB), _f32),
                        pltpu.VMEM((NB, R, EMB), _f32)],
    )(h_in, agg, batch3, vn, opa, W1, b1, W2, b2, vW1, vb1, vW2, vb2)


def _sc_mesh():
    return plsc.VectorSubcoreMesh(core_axis_name="c", subcore_axis_name="s")


def _zero_vmem_block(zv, rows):
    def zrow(r, carry):
        for col in range(EMB // 16):
            zv[r, pl.ds(col * 16, 16)] = jnp.zeros((16,), _f32)
        return carry
    lax.fori_loop(0, rows, zrow, 0)


def _zero_agg(zbuf, agg_s, s, rows):
    _zero_vmem_block(zbuf, rows)
    r0 = s * RPT
    for j in range(RPT // rows):
        pltpu.sync_copy(zbuf, agg_s.at[pl.ds(r0 + j * rows, rows)])
    plsc.subcore_barrier()


def _copy_out(agg_s, out_h, c, s, rows):
    plsc.subcore_barrier()
    r0 = s * RPT
    for j in range(RPT // rows):
        r = r0 + j * rows
        pltpu.sync_copy(agg_s.at[pl.ds(r, rows)],
                        out_h.at[pl.ds(c * NP + r, rows)])


def _sc_scatter_only(msg, dstp):
    @functools.partial(
        pl.kernel,
        mesh=_sc_mesh(),
        out_type=jax.ShapeDtypeStruct((NC * NP, EMB), _f32),
        scratch_types=[
            pltpu.VMEM((CHS,), jnp.int32),
            pltpu.VMEM((CHS,), jnp.int32),
            pltpu.VMEM((CHS, EMB), _f32),
            pltpu.VMEM((CHS, EMB), _f32),
            pltpu.VMEM_SHARED((NP, EMB), _f32),
            pltpu.SemaphoreType.DMA,
            pltpu.SemaphoreType.DMA,
        ],
    )
    def k(msg_h, dst_h, out_h, didx0, didx1, buf0, buf1, agg_s, lsem0, lsem1):
        c = lax.axis_index("c")
        s = lax.axis_index("s")
        wid = s * NC + c
        _zero_agg(buf0, agg_s, s, CHS)
        base = wid * WE
        didx = (didx0, didx1)
        buf = (buf0, buf1)
        lsem = (lsem0, lsem1)

        def issue(t, b):
            off = base + t * CHS
            pltpu.async_copy(dst_h.at[pl.ds(off, CHS)], didx[b], lsem[b])
            pltpu.async_copy(msg_h.at[pl.ds(off, CHS)], buf[b], lsem[b])

        def drain(b):
            pltpu.make_async_copy(dst_h.at[pl.ds(0, CHS)], didx[b],
                                  lsem[b]).wait()
            pltpu.make_async_copy(msg_h.at[pl.ds(0, CHS)], buf[b],
                                  lsem[b]).wait()

        issue(0, 0)

        def pair(i, carry):
            t1 = 2 * i + 1
            t2 = 2 * i + 2
            drain(0)
            issue(t1, 1)
            pltpu.sync_copy(buf[0], agg_s.at[didx[0]], add=True)
            drain(1)

            @pl.when(t2 < NCHS)
            def _():
                issue(t2, 0)

            pltpu.sync_copy(buf[1], agg_s.at[didx[1]], add=True)
            return carry

        lax.fori_loop(0, NCHS // 2, pair, 0)
        _copy_out(agg_s, out_h, c, s, CHS)

    return k(msg, dstp).reshape(NC, NP, EMB)


def _sc_edge(ee, srcp, dstp, h_in):
    @functools.partial(
        pl.kernel,
        mesh=_sc_mesh(),
        out_type=jax.ShapeDtypeStruct((NC * NP, EMB), _f32),
        scratch_types=[
            pltpu.VMEM((CHE,), jnp.int32),
            pltpu.VMEM((CHE,), jnp.int32),
            pltpu.VMEM((CHE,), jnp.int32),
            pltpu.VMEM((CHE,), jnp.int32),
            pltpu.VMEM((CHE, EMB), _f32),
            pltpu.VMEM((CHE, EMB), _f32),
            pltpu.VMEM((CHE, EMB), _f32),
            pltpu.VMEM((CHE, EMB), _f32),
            pltpu.VMEM_SHARED((NP, EMB), _f32),
            pltpu.SemaphoreType.DMA,
            pltpu.SemaphoreType.DMA,
            pltpu.SemaphoreType.DMA,
            pltpu.SemaphoreType.DMA,
        ],
    )
    def k(ee_h, src_h, dst_h, hin_h, out_h,
          sidx0, sidx1, didx0, didx1, ebuf0, ebuf1, hbuf0, hbuf1,
          agg_s, lsem0, lsem1, gsem0, gsem1):
        c = lax.axis_index("c")
        s = lax.axis_index("s")
        _zero_agg(ebuf0, agg_s, s, CHE)
        sidx = (sidx0, sidx1)
        didx = (didx0, didx1)
        ebuf = (ebuf0, ebuf1)
        hbuf = (hbuf0, hbuf1)
        lsem = (lsem0, lsem1)
        gsem = (gsem0, gsem1)

        def issue_loads(base, t, b):
            off = base + t * CHE
            pltpu.async_copy(src_h.at[pl.ds(off, CHE)], sidx[b], lsem[b])
            pltpu.async_copy(dst_h.at[pl.ds(off, CHE)], didx[b], lsem[b])
            pltpu.async_copy(ee_h.at[pl.ds(off, CHE)], ebuf[b], lsem[b])

        def drain_loads(b):
            pltpu.make_async_copy(src_h.at[pl.ds(0, CHE)], sidx[b],
                                  lsem[b]).wait()
            pltpu.make_async_copy(dst_h.at[pl.ds(0, CHE)], didx[b],
                                  lsem[b]).wait()
            pltpu.make_async_copy(ee_h.at[pl.ds(0, CHE)], ebuf[b],
                                  lsem[b]).wait()

        def issue_gather(b):
            pltpu.async_copy(hin_h.at[sidx[b]], hbuf[b], gsem[b])

        def drain_gather(b):
            pltpu.make_async_copy(hin_h.at[sidx[b]], hbuf[b],
                                  gsem[b]).wait()

        def compute(b):
            eb = ebuf[b]
            hb = hbuf[b]

            def row(r, cc):
                for u in range(2):
                    rr = r * 2 + u
                    for col in range(EMB // 16):
                        sl = (rr, pl.ds(col * 16, 16))
                        eb[sl] = jnp.maximum(eb[sl] + hb[sl], 0.0)
                return cc

            lax.fori_loop(0, CHE // 2, row, 0)

        def scatter(b):
            pltpu.sync_copy(ebuf[b], agg_s.at[didx[b]], add=True)

        def run(base, nch):
            issue_loads(base, 0, 0)
            drain_loads(0)
            issue_gather(0)

            def pair(i, carry):
                t1 = 2 * i + 1
                t2 = 2 * i + 2
                issue_loads(base, t1, 1)
                drain_gather(0)
                compute(0)
                drain_loads(1)
                issue_gather(1)
                scatter(0)
                @pl.when(t2 < nch)
                def _():
                    issue_loads(base, t2, 0)

                drain_gather(1)
                compute(1)

                @pl.when(t2 < nch)
                def _():
                    drain_loads(0)
                    issue_gather(0)

                scatter(1)
                return carry

            lax.fori_loop(0, nch // 2, pair, 0)

        @pl.when(c == 0)
        def _():
            run(s * K0 * CHE, K0)

        @pl.when(c == 1)
        def _():
            run((NS * K0 + s * K1) * CHE, K1)

        _copy_out(agg_s, out_h, c, s, CHS)

    return k(ee, srcp, dstp, h_in).reshape(NC, NP, EMB)


def kernel(params, edge_attr, x, edge_index, batch):
    p = params
    kbn = np.float32(1.0 / np.sqrt(1.0 + BN_EPS))

    c0 = p["node_enc"][0] + p["vn_emb"][0]
    vn0 = jnp.broadcast_to(p["vn_emb"][0], (NG, EMB))

    src = edge_index[0]
    dst = edge_index[1]
    pad = EP - E
    srcp = jnp.concatenate([src, jnp.zeros((pad,), jnp.int32)])
    dstp = jnp.concatenate([dst, jnp.full((pad,), N, jnp.int32)])
    attr8 = jnp.concatenate(
        [edge_attr, jnp.zeros((E, 1), _f32)], axis=1)
    attr8 = jnp.concatenate(
        [attr8, jnp.zeros((pad, 8), _f32)], axis=0)
    attr8 = attr8.at[:E, 7].set(1.0)
    batch3 = batch.reshape(NB, 1, R)

    Ls = []
    for l in range(3):
        q = p["layers"][l]
        g1s = q["g1"] * kbn
        W1 = q["W1"] * g1s[None, :]
        b1 = (q["b1"] * g1s + q["b1n"])[None, :]
        gos = p["bn_g"][l] * kbn
        W2 = q["W2"] * gos[None, :]
        b2 = (q["b2"] * gos + p["bn_b"][l])[None, :]
        bias = q["eeb"] + (c0 if l == 0 else 0.0)
        eeW = jnp.concatenate([q["eeW"], bias[None, :]], axis=0)
        opa = (1.0 + q["eps"]).reshape(1, 1)
        Ls.append((eeW, opa, W1, b1, W2, b2))
    Vs = []
    for l in range(2):
        m = p["vn_mlps"][l]
        gs1 = m["g1"] * kbn
        vW1 = m["W1"] * gs1[None, :]
        vb1 = (m["b1"] * gs1 + m["be1"])[None, :]
        gs2 = m["g2"] * kbn
        vW2 = m["W2"] * gs2[None, :]
        vb2 = (m["b2"] * gs2 + m["be2"])[None, :]
        Vs.append((vW1, vb1, vW2, vb2))

    msg0, ee1, ee2 = _tc_pre(attr8, Ls[0][0], Ls[1][0], Ls[2][0])

    agg0 = _sc_scatter_only(msg0, dstp)
    h_in1, vn1 = _dense_mid(c0[None], agg0, batch3, vn0, *Ls[0][1:], *Vs[0],
                            const_hin=True)

    agg1 = _sc_edge(ee1, srcp, dstp, h_in1)
    h_in2, vn2 = _dense_mid(h_in1, agg1, batch3, vn1, *Ls[1][1:], *Vs[1])

    agg2 = _sc_edge(ee2, srcp, dstp, h_in2)
    return _dense_last(h_in2, agg2, *Ls[2][1:])
